# Initial kernel scaffold; baseline (speedup 1.0000x reference)
import jax
import jax.numpy as jnp
from jax.experimental import pallas as pl


def kernel(x, codebook):
    raise NotImplementedError("write your pallas kernel here")


# speedup vs baseline: 1.0911x; 1.0911x over previous
import jax
import jax.numpy as jnp
from jax.experimental import pallas as pl
from jax.experimental.pallas import tpu as pltpu

_B = 8
_E = 256
_N = 1024
_HW = 1024
_COMMIT = 0.25
_LOSS_SCALE = (1.0 + _COMMIT) / float(_B * _E * _HW)


def _vq_body(x_ref, cb_ref, cbt_ref, q_ref, loss_ref):
    b = pl.program_id(0)
    xb = x_ref[0]
    cb = cb_ref[...]
    cbt = cbt_ref[...]

    m = jnp.dot(cb, xb, preferred_element_type=jnp.float32)
    cnorm = jnp.sum(cb * cb, axis=1, keepdims=True)
    xnorm = jnp.sum(xb * xb, axis=0, keepdims=True)
    dist = (cnorm + xnorm) - 2.0 * m
    idx = jnp.argmin(dist, axis=0)

    onehot = (jax.lax.broadcasted_iota(jnp.int32, (_N, _HW), 0)
              == idx[None, :]).astype(jnp.float32)
    q = jax.lax.dot(cbt, onehot,
                    precision=jax.lax.Precision.HIGHEST,
                    preferred_element_type=jnp.float32)
    q_ref[0] = q

    part = jnp.sum((q - xb) ** 2)

    @pl.when(b == 0)
    def _init():
        loss_ref[0, 0] = 0.0

    loss_ref[0, 0] += part

    @pl.when(b == _B - 1)
    def _fin():
        loss_ref[0, 0] = loss_ref[0, 0] * _LOSS_SCALE


def kernel(x, codebook):
    xr = x.reshape(_B, _E, _HW)
    cbt = codebook.T
    q, loss = pl.pallas_call(
        _vq_body,
        grid=(_B,),
        in_specs=[
            pl.BlockSpec((1, _E, _HW), lambda b: (b, 0, 0)),
            pl.BlockSpec((_N, _E), lambda b: (0, 0)),
            pl.BlockSpec((_E, _N), lambda b: (0, 0)),
        ],
        out_specs=[
            pl.BlockSpec((1, _E, _HW), lambda b: (b, 0, 0)),
            pl.BlockSpec(memory_space=pltpu.SMEM,
                         block_shape=(1, 1), index_map=lambda b: (0, 0)),
        ],
        out_shape=[
            jax.ShapeDtypeStruct((_B, _E, _HW), jnp.float32),
            jax.ShapeDtypeStruct((1, 1), jnp.float32),
        ],
    )(xr, codebook, cbt)
    return (q.reshape(_B, _E, 32, 32), loss[0, 0])


# speedup vs baseline: 1.5350x; 1.4068x over previous
import jax
import jax.numpy as jnp
from jax.experimental import pallas as pl
from jax.experimental.pallas import tpu as pltpu

_B = 8
_E = 256
_N = 1024
_HW = 1024
_COMMIT = 0.25
_LOSS_SCALE = (1.0 + _COMMIT) / float(_B * _E * _HW)


def _vq_body(x_ref, cb_ref, cbt_hi_ref, cbt_lo_ref, q_ref, loss_ref):
    b = pl.program_id(0)
    xb = x_ref[0]
    cb = cb_ref[...]

    m = jnp.dot(cb, xb, preferred_element_type=jnp.float32)
    cnorm = jnp.sum(cb * cb, axis=1, keepdims=True)
    xnorm = jnp.sum(xb * xb, axis=0, keepdims=True)
    dist = (cnorm + xnorm) - 2.0 * m
    idx = jnp.argmin(dist, axis=0)

    onehot = (jax.lax.broadcasted_iota(jnp.int32, (_N, _HW), 0)
              == idx[None, :]).astype(jnp.bfloat16)
    q = (jnp.dot(cbt_hi_ref[...], onehot, preferred_element_type=jnp.float32)
         + jnp.dot(cbt_lo_ref[...], onehot,
                   preferred_element_type=jnp.float32))
    q_ref[0] = q

    part = jnp.sum((q - xb) ** 2)

    @pl.when(b == 0)
    def _init():
        loss_ref[0, 0] = 0.0

    loss_ref[0, 0] += part

    @pl.when(b == _B - 1)
    def _fin():
        loss_ref[0, 0] = loss_ref[0, 0] * _LOSS_SCALE


def kernel(x, codebook):
    xr = x.reshape(_B, _E, _HW)
    cbt = codebook.T
    cbt_hi = cbt.astype(jnp.bfloat16)
    cbt_lo = (cbt - cbt_hi.astype(jnp.float32)).astype(jnp.bfloat16)
    q, loss = pl.pallas_call(
        _vq_body,
        grid=(_B,),
        in_specs=[
            pl.BlockSpec((1, _E, _HW), lambda b: (b, 0, 0)),
            pl.BlockSpec((_N, _E), lambda b: (0, 0)),
            pl.BlockSpec((_E, _N), lambda b: (0, 0)),
            pl.BlockSpec((_E, _N), lambda b: (0, 0)),
        ],
        out_specs=[
            pl.BlockSpec((1, _E, _HW), lambda b: (b, 0, 0)),
            pl.BlockSpec(memory_space=pltpu.SMEM,
                         block_shape=(1, 1), index_map=lambda b: (0, 0)),
        ],
        out_shape=[
            jax.ShapeDtypeStruct((_B, _E, _HW), jnp.float32),
            jax.ShapeDtypeStruct((1, 1), jnp.float32),
        ],
    )(xr, codebook, cbt_hi, cbt_lo)
    return (q.reshape(_B, _E, 32, 32), loss[0, 0])


# speedup vs baseline: 1.5573x; 1.0145x over previous
import jax
import jax.numpy as jnp
from jax.experimental import pallas as pl
from jax.experimental.pallas import tpu as pltpu

_B = 8
_E = 256
_N = 1024
_HW = 1024
_COMMIT = 0.25
_LOSS_SCALE = (1.0 + _COMMIT) / float(_B * _E * _HW)


def _vq_body(x_ref, cb_ref, cbt_hi_ref, cbt_lo_ref, q_ref, loss_ref):
    b = pl.program_id(0)
    xb = x_ref[0]
    cb = cb_ref[...]

    m = jnp.dot(cb, xb, preferred_element_type=jnp.float32)
    cnorm = jnp.sum(cb * cb, axis=1, keepdims=True)
    xnorm = jnp.sum(xb * xb, axis=0, keepdims=True)
    dist = (cnorm + xnorm) - 2.0 * m
    idx = jnp.argmin(dist, axis=0)

    eq16 = (jax.lax.broadcasted_iota(jnp.int16, (_N, _HW), 0)
            == idx.astype(jnp.int16)[None, :])
    onehot = jnp.where(eq16, jnp.bfloat16(1), jnp.bfloat16(0))
    q = (jnp.dot(cbt_hi_ref[...], onehot, preferred_element_type=jnp.float32)
         + jnp.dot(cbt_lo_ref[...], onehot,
                   preferred_element_type=jnp.float32))
    q_ref[0] = q

    part = jnp.sum((q - xb) ** 2)

    @pl.when(b == 0)
    def _init():
        loss_ref[0, 0] = 0.0

    loss_ref[0, 0] += part

    @pl.when(b == _B - 1)
    def _fin():
        loss_ref[0, 0] = loss_ref[0, 0] * _LOSS_SCALE


def kernel(x, codebook):
    xr = x.reshape(_B, _E, _HW)
    cbt = codebook.T
    cbt_hi = cbt.astype(jnp.bfloat16)
    cbt_lo = (cbt - cbt_hi.astype(jnp.float32)).astype(jnp.bfloat16)
    q, loss = pl.pallas_call(
        _vq_body,
        grid=(_B,),
        in_specs=[
            pl.BlockSpec((1, _E, _HW), lambda b: (b, 0, 0)),
            pl.BlockSpec((_N, _E), lambda b: (0, 0)),
            pl.BlockSpec((_E, _N), lambda b: (0, 0)),
            pl.BlockSpec((_E, _N), lambda b: (0, 0)),
        ],
        out_specs=[
            pl.BlockSpec((1, _E, _HW), lambda b: (b, 0, 0)),
            pl.BlockSpec(memory_space=pltpu.SMEM,
                         block_shape=(1, 1), index_map=lambda b: (0, 0)),
        ],
        out_shape=[
            jax.ShapeDtypeStruct((_B, _E, _HW), jnp.float32),
            jax.ShapeDtypeStruct((1, 1), jnp.float32),
        ],
    )(xr, codebook, cbt_hi, cbt_lo)
    return (q.reshape(_B, _E, 32, 32), loss[0, 0])


# speedup vs baseline: 1.6012x; 1.0281x over previous
import jax
import jax.numpy as jnp
from jax.experimental import pallas as pl
from jax.experimental.pallas import tpu as pltpu

_B = 8
_E = 256
_N = 1024
_HW = 1024
_COMMIT = 0.25
_LOSS_SCALE = (1.0 + _COMMIT) / float(_B * _E * _HW)


def _vq_body(x_ref, cb_ref, cbt_hi_ref, cbt_lo_ref, q_ref, loss_ref):
    b = pl.program_id(0)
    cb = cb_ref[...]
    for s in range(2):
        _vq_one(s, b, x_ref, cb, cbt_hi_ref, cbt_lo_ref, q_ref, loss_ref)


def _vq_one(s, b, x_ref, cb, cbt_hi_ref, cbt_lo_ref, q_ref, loss_ref):
    xb = x_ref[s]

    m = jnp.dot(cb, xb, preferred_element_type=jnp.float32)
    cnorm = jnp.sum(cb * cb, axis=1, keepdims=True)
    xnorm = jnp.sum(xb * xb, axis=0, keepdims=True)
    dist = (cnorm + xnorm) - 2.0 * m
    idx = jnp.argmin(dist, axis=0)

    eq16 = (jax.lax.broadcasted_iota(jnp.int16, (_N, _HW), 0)
            == idx.astype(jnp.int16)[None, :])
    onehot = jnp.where(eq16, jnp.bfloat16(1), jnp.bfloat16(0))
    q = (jnp.dot(cbt_hi_ref[...], onehot, preferred_element_type=jnp.float32)
         + jnp.dot(cbt_lo_ref[...], onehot,
                   preferred_element_type=jnp.float32))
    q_ref[s] = q

    part = jnp.sum((q - xb) ** 2)

    @pl.when(jnp.logical_and(b == 0, s == 0))
    def _init():
        loss_ref[0, 0] = 0.0

    loss_ref[0, 0] += part

    @pl.when(jnp.logical_and(b == _B // 2 - 1, s == 1))
    def _fin():
        loss_ref[0, 0] = loss_ref[0, 0] * _LOSS_SCALE


def kernel(x, codebook):
    xr = x.reshape(_B, _E, _HW)
    cbt = codebook.T
    cbt_hi = cbt.astype(jnp.bfloat16)
    cbt_lo = (cbt - cbt_hi.astype(jnp.float32)).astype(jnp.bfloat16)
    q, loss = pl.pallas_call(
        _vq_body,
        grid=(_B // 2,),
        in_specs=[
            pl.BlockSpec((2, _E, _HW), lambda b: (b, 0, 0)),
            pl.BlockSpec((_N, _E), lambda b: (0, 0)),
            pl.BlockSpec((_E, _N), lambda b: (0, 0)),
            pl.BlockSpec((_E, _N), lambda b: (0, 0)),
        ],
        out_specs=[
            pl.BlockSpec((2, _E, _HW), lambda b: (b, 0, 0)),
            pl.BlockSpec(memory_space=pltpu.SMEM,
                         block_shape=(1, 1), index_map=lambda b: (0, 0)),
        ],
        out_shape=[
            jax.ShapeDtypeStruct((_B, _E, _HW), jnp.float32),
            jax.ShapeDtypeStruct((1, 1), jnp.float32),
        ],
    )(xr, codebook, cbt_hi, cbt_lo)
    return (q.reshape(_B, _E, 32, 32), loss[0, 0])
